# baseline (speedup 1.0000x reference)
import jax
import jax.numpy as jnp
from jax.experimental import pallas as pl
from jax.experimental.pallas import tpu as pltpu

_ROWS = 8


def _max_kernel(x_ref, out_ref):
    out_ref[0, 0, :] = jnp.max(x_ref[...], axis=1)


def _mask_kernel(v_ref, m_ref, *, k):
    v = v_ref[...]
    b_, c_ = v.shape
    vj = v[:, None, :]
    vc = v[:, :, None]
    j = jax.lax.broadcasted_iota(jnp.int32, (1, c_, c_), 2)
    c = jax.lax.broadcasted_iota(jnp.int32, (1, c_, c_), 1)
    beats = (vj > vc) | ((vj == vc) & (j < c))
    rank = jnp.sum(beats.astype(jnp.int32), axis=2)
    m_ref[...] = (rank < k).astype(v.dtype)


def _mul_kernel(x_ref, m_ref, out_ref):
    m = m_ref[0, 0, :]
    out_ref[...] = x_ref[...] * m[:, None]


def kernel(x):
    B, C, H, W = x.shape
    HW = H * W
    k = C // 2
    rows = B * C
    nblk = rows // _ROWS

    x2 = x.reshape(rows, HW)

    maxes = pl.pallas_call(
        _max_kernel,
        grid=(nblk,),
        in_specs=[pl.BlockSpec((_ROWS, HW), lambda i: (i, 0))],
        out_specs=pl.BlockSpec((1, 1, _ROWS), lambda i: (i, 0, 0)),
        out_shape=jax.ShapeDtypeStruct((nblk, 1, _ROWS), x.dtype),
    )(x2)

    v = maxes.reshape(B, C)

    mask = pl.pallas_call(
        lambda v_ref, m_ref: _mask_kernel(v_ref, m_ref, k=k),
        in_specs=[pl.BlockSpec((B, C), lambda: (0, 0))],
        out_specs=pl.BlockSpec((B, C), lambda: (0, 0)),
        out_shape=jax.ShapeDtypeStruct((B, C), x.dtype),
    )(v)

    m3 = mask.reshape(nblk, 1, _ROWS)

    out = pl.pallas_call(
        _mul_kernel,
        grid=(nblk,),
        in_specs=[
            pl.BlockSpec((_ROWS, HW), lambda i: (i, 0)),
            pl.BlockSpec((1, 1, _ROWS), lambda i: (i, 0, 0)),
        ],
        out_specs=pl.BlockSpec((_ROWS, HW), lambda i: (i, 0)),
        out_shape=jax.ShapeDtypeStruct((rows, HW), x.dtype),
    )(x2, m3)

    return out.reshape(B, C, H, W)


# speedup vs baseline: 3.1108x; 3.1108x over previous
import jax
import jax.numpy as jnp
from jax.experimental import pallas as pl
from jax.experimental.pallas import tpu as pltpu

_CB = 16


def _max_kernel(x_ref, out_ref):
    out_ref[0, 0, :] = jnp.max(x_ref[...], axis=(0, 2, 3))


def _mask_kernel(v_ref, m_ref, *, k):
    v = v_ref[...]
    b_, c_ = v.shape
    vj = v[:, None, :]
    vc = v[:, :, None]
    j = jax.lax.broadcasted_iota(jnp.int32, (1, c_, c_), 2)
    c = jax.lax.broadcasted_iota(jnp.int32, (1, c_, c_), 1)
    beats = (vj > vc) | ((vj == vc) & (j < c))
    rank = jnp.sum(beats.astype(jnp.int32), axis=2)
    m_ref[...] = (rank < k).astype(v.dtype)


def _mul_kernel(x_ref, m_ref, out_ref):
    m = m_ref[0, 0, :]
    out_ref[...] = x_ref[...] * m[None, :, None, None]


def kernel(x):
    B, C, H, W = x.shape
    k = C // 2
    ncb = C // _CB

    maxes = pl.pallas_call(
        _max_kernel,
        grid=(B, ncb),
        in_specs=[pl.BlockSpec((1, _CB, H, W), lambda b, i: (b, i, 0, 0))],
        out_specs=pl.BlockSpec((1, 1, _CB), lambda b, i: (b * ncb + i, 0, 0)),
        out_shape=jax.ShapeDtypeStruct((B * ncb, 1, _CB), x.dtype),
    )(x)

    v = maxes.reshape(B, C)

    mask = pl.pallas_call(
        lambda v_ref, m_ref: _mask_kernel(v_ref, m_ref, k=k),
        in_specs=[pl.BlockSpec((B, C), lambda: (0, 0))],
        out_specs=pl.BlockSpec((B, C), lambda: (0, 0)),
        out_shape=jax.ShapeDtypeStruct((B, C), x.dtype),
    )(v)

    m3 = mask.reshape(B * ncb, 1, _CB)

    out = pl.pallas_call(
        _mul_kernel,
        grid=(B, ncb),
        in_specs=[
            pl.BlockSpec((1, _CB, H, W), lambda b, i: (b, i, 0, 0)),
            pl.BlockSpec((1, 1, _CB), lambda b, i: (b * ncb + i, 0, 0)),
        ],
        out_specs=pl.BlockSpec((1, _CB, H, W), lambda b, i: (b, i, 0, 0)),
        out_shape=jax.ShapeDtypeStruct((B, C, H, W), x.dtype),
    )(x, m3)

    return out
